# baseline (speedup 1.0000x reference)
import jax
import jax.numpy as jnp
from jax import lax
from jax.experimental import pallas as pl
from jax.experimental.pallas import tpu as pltpu
from jax.experimental.pallas import tpu_sc as plsc

N = 10000
CHUNK = 40
NUM_CHUNKS = N // CHUNK
NW = 32
SLOTS = (NUM_CHUNKS + NW - 1) // NW
LANES = 16
IMG_SIZES = (56, 28, 14, 7)
CHANNELS = (64, 128, 256, 512)
COL_OFF = (0, 64, 192, 448)
OUT_COLS = 960
ROWS_PER_W = SLOTS * CHUNK


def _prep_body(f0, f1, f2, f3, t0, t1, t2, t3):
    for f, t, size, ch in zip((f0, f1, f2, f3), (t0, t1, t2, t3),
                              IMG_SIZES, CHANNELS):
        x = f[...].reshape(ch, size * size)
        t[0:size * size, :] = x.T
        t[size * size:size * size + 1, :] = jnp.zeros((1, ch), jnp.float32)


def _make_tables(feats, batch):
    ins = [f[batch] for f in feats]
    return pl.pallas_call(
        _prep_body,
        out_shape=[jax.ShapeDtypeStruct((size * size + 1, ch), jnp.float32)
                   for size, ch in zip(IMG_SIZES, CHANNELS)],
    )(*ins)


def _body(t0, t1, t2, t3, inp, out,
          vbuf, i0, i1, i2, i3,
          r00, r01, r02, r03, r10, r11, r12, r13,
          st0, st1, st2, st3,
          isem, gs0, gs1, os0, os1):
    tabs = (t0, t1, t2, t3)
    idxs = (i0, i1, i2, i3)
    rows = ((r00, r01, r02, r03), (r10, r11, r12, r13))
    stabs = (st0, st1, st2, st3)
    gsems = (gs0, gs1)
    osems = (os0, os1)
    sid = lax.axis_index("s")
    wid = sid * 2 + lax.axis_index("c")

    bases = []
    handles = []
    for j in range(SLOTS):
        c = wid + NW * j
        c = jnp.where(c < NUM_CHUNKS, c, wid)
        base = c * CHUNK
        bases.append(base)
        handles.append(pltpu.async_copy(
            inp.at[pl.ds(base, CHUNK), :],
            vbuf.at[pl.ds(j * CHUNK, CHUNK), :], isem))

    for s in range(4):
        @pl.when(sid == s)
        def _():
            pltpu.sync_copy(tabs[s], stabs[s])

    for h in handles:
        h.wait()

    for i in range(ROWS_PER_W // LANES):
        sl = pl.ds(i * LANES, LANES)
        rvec = lax.iota(jnp.int32, 16) + (i * LANES)
        zc = jnp.zeros((16,), jnp.int32)
        a0 = plsc.load_gather(vbuf, [rvec, zc])
        a1 = plsc.load_gather(vbuf, [rvec, zc + 1])
        a2 = plsc.load_gather(vbuf, [rvec, zc + 2])
        h = 248.0 * (a1 / a2) + 111.5
        w = 248.0 * (a0 / (-a2)) + 111.5
        h = jnp.minimum(jnp.maximum(h, 0.0), 223.0)
        w = jnp.minimum(jnp.maximum(w, 0.0), 223.0)
        for s, size in enumerate(IMG_SIZES):
            x = h * (size / 224.0)
            y = w * (size / 224.0)
            xi = x.astype(jnp.int32)
            yi = y.astype(jnp.int32)
            xi = jnp.minimum(jnp.maximum(xi, 0), size - 1)
            yi = jnp.minimum(jnp.maximum(yi, 0), size - 1)
            ok = ((x > xi.astype(jnp.float32))
                  & (y > yi.astype(jnp.float32))
                  & (xi < size - 1) & (yi < size - 1))
            idx = xi * size + yi
            idxs[s][sl] = jnp.where(ok, idx, size * size)

    plsc.subcore_barrier()

    def fire_gathers(j, p):
        return [pltpu.async_copy(
                    stabs[s].at[idxs[s].at[pl.ds(j * CHUNK, CHUNK)]],
                    rows[p][s], gsems[p])
                for s in range(4)]

    def fire_outs(j, p):
        return [pltpu.async_copy(
                    rows[p][s],
                    out.at[pl.ds(bases[j], CHUNK),
                           pl.ds(COL_OFF[s], CHANNELS[s])],
                    osems[p])
                for s in range(4)]

    pend_g = {0: fire_gathers(0, 0), 1: None}
    pend_o = {0: None, 1: None}
    for j in range(SLOTS):
        p = j & 1
        q = 1 - p
        if j + 1 < SLOTS:
            if pend_o[q] is not None:
                for h in pend_o[q]:
                    h.wait()
            pend_g[q] = fire_gathers(j + 1, q)
        for h in pend_g[p]:
            h.wait()
        pend_o[p] = fire_outs(j, p)
    for p in range(2):
        if pend_o[p] is not None:
            for h in pend_o[p]:
                h.wait()


def kernel(img_feat0, img_feat1, img_feat2, img_feat3, input, batch):
    tables = _make_tables((img_feat0, img_feat1, img_feat2, img_feat3), batch)

    mesh = plsc.VectorSubcoreMesh(core_axis_name="c", subcore_axis_name="s")
    scratch = (
        [pltpu.VMEM((ROWS_PER_W, 3), jnp.float32)]
        + [pltpu.VMEM((ROWS_PER_W,), jnp.int32) for _ in range(4)]
        + [pltpu.VMEM((CHUNK, ch), jnp.float32)
           for _ in range(2) for ch in CHANNELS]
        + [pltpu.VMEM_SHARED((size * size + 1, ch), jnp.float32)
           for size, ch in zip(IMG_SIZES, CHANNELS)]
        + [pltpu.SemaphoreType.DMA] * 5
    )
    run = pl.kernel(
        _body,
        out_type=jax.ShapeDtypeStruct((N, OUT_COLS), jnp.float32),
        mesh=mesh,
        scratch_types=scratch,
        compiler_params=pltpu.CompilerParams(use_tc_tiling_on_sc=False,
                                             needs_layout_passes=False),
    )
    return run(*tables, input)


# speedup vs baseline: 1.8728x; 1.8728x over previous
import jax
import jax.numpy as jnp
from jax import lax
from jax.experimental import pallas as pl
from jax.experimental.pallas import tpu as pltpu
from jax.experimental.pallas import tpu_sc as plsc

N = 10000
CHUNK = 40
NUM_CHUNKS = N // CHUNK
NW = 32
SLOTS = (NUM_CHUNKS + NW - 1) // NW
LANES = 16
IMG_SIZES = (56, 28, 14, 7)
CHANNELS = (64, 128, 256, 512)
COL_OFF = (0, 64, 192, 448)
OUT_COLS = 960
ROWS_PER_W = SLOTS * CHUNK


RB = 50


def _retile_body(l_ref, o_ref):
    for t in range(8):
        x = l_ref[:, :, t, :].reshape(RB * 8, 128)
        if 128 * (t + 1) <= OUT_COLS:
            o_ref[:, 128 * t:128 * (t + 1)] = x
        else:
            o_ref[:, 128 * t:OUT_COLS] = x[:, :OUT_COLS - 128 * t]


def _retile(l_flat):
    l4 = l_flat.reshape(N // 8, 8, 8, 128)
    return pl.pallas_call(
        _retile_body,
        grid=(N // 8 // RB,),
        in_specs=[pl.BlockSpec((RB, 8, 8, 128), lambda g: (g, 0, 0, 0))],
        out_specs=pl.BlockSpec((RB * 8, OUT_COLS), lambda g: (g, 0)),
        out_shape=jax.ShapeDtypeStruct((N, OUT_COLS), jnp.float32),
    )(l4)


def _prep_body(f0, f1, f2, f3, t0, t1, t2, t3):
    for f, t, size, ch in zip((f0, f1, f2, f3), (t0, t1, t2, t3),
                              IMG_SIZES, CHANNELS):
        x = f[...].reshape(ch, size * size)
        t[0:size * size, :] = x.T
        t[size * size:size * size + 1, :] = jnp.zeros((1, ch), jnp.float32)


def _make_tables(feats, batch):
    ins = [f[batch] for f in feats]
    return pl.pallas_call(
        _prep_body,
        out_shape=[jax.ShapeDtypeStruct((size * size + 1, ch), jnp.float32)
                   for size, ch in zip(IMG_SIZES, CHANNELS)],
    )(*ins)


def _body(t0, t1, t2, t3, inp, out,
          vbuf, i0, i1, i2, i3,
          r00, r01, r02, r03, r10, r11, r12, r13,
          st0, st1, st2, st3,
          isem, gs0, gs1, os0, os1):
    tabs = (t0, t1, t2, t3)
    idxs = (i0, i1, i2, i3)
    rows = ((r00, r01, r02, r03), (r10, r11, r12, r13))
    stabs = (st0, st1, st2, st3)
    gsems = (gs0, gs1)
    osems = (os0, os1)
    sid = lax.axis_index("s")
    wid = sid * 2 + lax.axis_index("c")

    bases = []
    handles = []
    for j in range(SLOTS):
        c = wid + NW * j
        c = jnp.where(c < NUM_CHUNKS, c, wid)
        base = c * CHUNK
        bases.append(base)
        handles.append(pltpu.async_copy(
            inp.at[pl.ds(base, CHUNK), :],
            vbuf.at[pl.ds(j * CHUNK, CHUNK), :], isem))

    for s in range(4):
        @pl.when(sid == s)
        def _():
            pltpu.sync_copy(tabs[s], stabs[s])

    for h in handles:
        h.wait()

    for i in range(ROWS_PER_W // LANES):
        sl = pl.ds(i * LANES, LANES)
        rvec = lax.iota(jnp.int32, 16) + (i * LANES)
        zc = jnp.zeros((16,), jnp.int32)
        a0 = plsc.load_gather(vbuf, [rvec, zc])
        a1 = plsc.load_gather(vbuf, [rvec, zc + 1])
        a2 = plsc.load_gather(vbuf, [rvec, zc + 2])
        h = 248.0 * (a1 / a2) + 111.5
        w = 248.0 * (a0 / (-a2)) + 111.5
        h = jnp.minimum(jnp.maximum(h, 0.0), 223.0)
        w = jnp.minimum(jnp.maximum(w, 0.0), 223.0)
        for s, size in enumerate(IMG_SIZES):
            x = h * (size / 224.0)
            y = w * (size / 224.0)
            xi = x.astype(jnp.int32)
            yi = y.astype(jnp.int32)
            xi = jnp.minimum(jnp.maximum(xi, 0), size - 1)
            yi = jnp.minimum(jnp.maximum(yi, 0), size - 1)
            ok = ((x > xi.astype(jnp.float32))
                  & (y > yi.astype(jnp.float32))
                  & (xi < size - 1) & (yi < size - 1))
            idx = xi * size + yi
            idxs[s][sl] = jnp.where(ok, idx, size * size)

    plsc.subcore_barrier()

    def fire_gathers(j, p):
        return [pltpu.async_copy(
                    stabs[s].at[idxs[s].at[pl.ds(j * CHUNK, CHUNK)]],
                    rows[p][s], gsems[p])
                for s in range(4)]

    def fire_outs(j, p):
        return [pltpu.async_copy(
                    rows[p][s],
                    out.at[pl.ds(bases[j], CHUNK),
                           pl.ds(COL_OFF[s], CHANNELS[s])],
                    osems[p])
                for s in range(4)]

    pend_g = {0: fire_gathers(0, 0), 1: None}
    pend_o = {0: None, 1: None}
    for j in range(SLOTS):
        p = j & 1
        q = 1 - p
        if j + 1 < SLOTS:
            if pend_o[q] is not None:
                for h in pend_o[q]:
                    h.wait()
            pend_g[q] = fire_gathers(j + 1, q)
        for h in pend_g[p]:
            h.wait()
        pend_o[p] = fire_outs(j, p)
    for p in range(2):
        if pend_o[p] is not None:
            for h in pend_o[p]:
                h.wait()


def kernel(img_feat0, img_feat1, img_feat2, img_feat3, input, batch):
    tables = _make_tables((img_feat0, img_feat1, img_feat2, img_feat3), batch)

    mesh = plsc.VectorSubcoreMesh(core_axis_name="c", subcore_axis_name="s")
    scratch = (
        [pltpu.VMEM((ROWS_PER_W, 3), jnp.float32)]
        + [pltpu.VMEM((ROWS_PER_W,), jnp.int32) for _ in range(4)]
        + [pltpu.VMEM((CHUNK, ch), jnp.float32)
           for _ in range(2) for ch in CHANNELS]
        + [pltpu.VMEM_SHARED((size * size + 1, ch), jnp.float32)
           for size, ch in zip(IMG_SIZES, CHANNELS)]
        + [pltpu.SemaphoreType.DMA] * 5
    )
    run = pl.kernel(
        _body,
        out_type=jax.ShapeDtypeStruct((N, 1024), jnp.float32),
        mesh=mesh,
        scratch_types=scratch,
        compiler_params=pltpu.CompilerParams(use_tc_tiling_on_sc=False,
                                             needs_layout_passes=False),
    )
    return _retile(run(*tables, input))


# speedup vs baseline: 2.6958x; 1.4394x over previous
import jax
import jax.numpy as jnp
from jax import lax
from jax.experimental import pallas as pl
from jax.experimental.pallas import tpu as pltpu
from jax.experimental.pallas import tpu_sc as plsc

N = 10000
CHUNK = 40
NUM_CHUNKS = N // CHUNK
NW = 32
SLOTS = (NUM_CHUNKS + NW - 1) // NW
LANES = 16
IMG_SIZES = (56, 28, 14, 7)
CHANNELS = (64, 128, 256, 512)
COL_OFF = (0, 64, 192, 448)
OUT_COLS = 960
ROWS_PER_W = SLOTS * CHUNK


BV = 1024


def _retile_body(l_ref, o_ref):
    for t in range(8):
        x = l_ref[:, :, t, :].reshape(BV, 128)
        xt = x.T
        lo = 128 * t
        hi = min(128 * (t + 1), OUT_COLS)
        o_ref[lo:hi, :] = xt[:hi - lo, :]


def _retile(l_flat):
    l4 = l_flat.reshape(N // 8, 8, 8, 128)
    return pl.pallas_call(
        _retile_body,
        grid=(pl.cdiv(N, BV),),
        in_specs=[pl.BlockSpec((BV // 8, 8, 8, 128), lambda g: (g, 0, 0, 0))],
        out_specs=pl.BlockSpec((OUT_COLS, BV), lambda g: (0, g)),
        out_shape=jax.ShapeDtypeStruct((OUT_COLS, N), jnp.float32),
    )(l4)


def _prep_body(f0, f1, f2, f3, t0, t1, t2, t3):
    for f, t, size, ch in zip((f0, f1, f2, f3), (t0, t1, t2, t3),
                              IMG_SIZES, CHANNELS):
        x = f[...].reshape(ch, size * size)
        t[0:size * size, :] = x.T
        t[size * size:size * size + 1, :] = jnp.zeros((1, ch), jnp.float32)


def _make_tables(feats, batch):
    ins = [f[batch] for f in feats]
    return pl.pallas_call(
        _prep_body,
        out_shape=[jax.ShapeDtypeStruct((size * size + 1, ch), jnp.float32)
                   for size, ch in zip(IMG_SIZES, CHANNELS)],
    )(*ins)


def _body(t0, t1, t2, t3, inp, out,
          vbuf, i0, i1, i2, i3,
          r00, r01, r02, r03, r10, r11, r12, r13,
          st0, st1, st2, st3,
          isem, gs0, gs1, os0, os1):
    tabs = (t0, t1, t2, t3)
    idxs = (i0, i1, i2, i3)
    rows = ((r00, r01, r02, r03), (r10, r11, r12, r13))
    stabs = (st0, st1, st2, st3)
    gsems = (gs0, gs1)
    osems = (os0, os1)
    sid = lax.axis_index("s")
    wid = sid * 2 + lax.axis_index("c")

    bases = []
    handles = []
    for j in range(SLOTS):
        c = wid + NW * j
        c = jnp.where(c < NUM_CHUNKS, c, wid)
        base = c * CHUNK
        bases.append(base)
        handles.append(pltpu.async_copy(
            inp.at[pl.ds(base, CHUNK), :],
            vbuf.at[pl.ds(j * CHUNK, CHUNK), :], isem))

    for s in range(4):
        @pl.when(sid == s)
        def _():
            pltpu.sync_copy(tabs[s], stabs[s])

    for h in handles:
        h.wait()

    for i in range(ROWS_PER_W // LANES):
        sl = pl.ds(i * LANES, LANES)
        rvec = lax.iota(jnp.int32, 16) + (i * LANES)
        zc = jnp.zeros((16,), jnp.int32)
        a0 = plsc.load_gather(vbuf, [rvec, zc])
        a1 = plsc.load_gather(vbuf, [rvec, zc + 1])
        a2 = plsc.load_gather(vbuf, [rvec, zc + 2])
        h = 248.0 * (a1 / a2) + 111.5
        w = 248.0 * (a0 / (-a2)) + 111.5
        h = jnp.minimum(jnp.maximum(h, 0.0), 223.0)
        w = jnp.minimum(jnp.maximum(w, 0.0), 223.0)
        for s, size in enumerate(IMG_SIZES):
            x = h * (size / 224.0)
            y = w * (size / 224.0)
            xi = x.astype(jnp.int32)
            yi = y.astype(jnp.int32)
            xi = jnp.minimum(jnp.maximum(xi, 0), size - 1)
            yi = jnp.minimum(jnp.maximum(yi, 0), size - 1)
            ok = ((x > xi.astype(jnp.float32))
                  & (y > yi.astype(jnp.float32))
                  & (xi < size - 1) & (yi < size - 1))
            idx = xi * size + yi
            idxs[s][sl] = jnp.where(ok, idx, size * size)

    plsc.subcore_barrier()

    def fire_gathers(j, p):
        return [pltpu.async_copy(
                    stabs[s].at[idxs[s].at[pl.ds(j * CHUNK, CHUNK)]],
                    rows[p][s], gsems[p])
                for s in range(4)]

    def fire_outs(j, p):
        return [pltpu.async_copy(
                    rows[p][s],
                    out.at[pl.ds(bases[j], CHUNK),
                           pl.ds(COL_OFF[s], CHANNELS[s])],
                    osems[p])
                for s in range(4)]

    pend_g = {0: fire_gathers(0, 0), 1: None}
    pend_o = {0: None, 1: None}
    for j in range(SLOTS):
        p = j & 1
        q = 1 - p
        if j + 1 < SLOTS:
            if pend_o[q] is not None:
                for h in pend_o[q]:
                    h.wait()
            pend_g[q] = fire_gathers(j + 1, q)
        for h in pend_g[p]:
            h.wait()
        pend_o[p] = fire_outs(j, p)
    for p in range(2):
        if pend_o[p] is not None:
            for h in pend_o[p]:
                h.wait()


def kernel(img_feat0, img_feat1, img_feat2, img_feat3, input, batch):
    tables = _make_tables((img_feat0, img_feat1, img_feat2, img_feat3), batch)

    mesh = plsc.VectorSubcoreMesh(core_axis_name="c", subcore_axis_name="s")
    scratch = (
        [pltpu.VMEM((ROWS_PER_W, 3), jnp.float32)]
        + [pltpu.VMEM((ROWS_PER_W,), jnp.int32) for _ in range(4)]
        + [pltpu.VMEM((CHUNK, ch), jnp.float32)
           for _ in range(2) for ch in CHANNELS]
        + [pltpu.VMEM_SHARED((size * size + 1, ch), jnp.float32)
           for size, ch in zip(IMG_SIZES, CHANNELS)]
        + [pltpu.SemaphoreType.DMA] * 5
    )
    run = pl.kernel(
        _body,
        out_type=jax.ShapeDtypeStruct((N, 1024), jnp.float32),
        mesh=mesh,
        scratch_types=scratch,
        compiler_params=pltpu.CompilerParams(use_tc_tiling_on_sc=False,
                                             needs_layout_passes=False),
    )
    return _retile(run(*tables, input)).T
